# baseline (speedup 1.0000x reference)
import functools

import numpy as np
import jax
import jax.numpy as jnp
from jax import lax
from jax.experimental import pallas as pl
from jax.experimental.pallas import tpu as pltpu
from jax.experimental.pallas import tpu_sc as plsc

N = 10000
K = 32
G = 16
DM = 128
HEADS = 8
FEAT = 16
OUT = 10

NP = 10240
TR = 256
CW = 512
WCAP = 4096
NT_KNN = NP // TR
MAXC = WCAP // CW

TRE = 512
TNP = 128
NW = 32
CH = 128
BPAD = NP * K
EP = BPAD // NW
NCH = EP // CH

_INF = float("inf")
_BIGI = 2**30


def _ln(h, g, b):
    m = jnp.mean(h, axis=-1, keepdims=True)
    v = jnp.mean((h - m) * (h - m), axis=-1, keepdims=True)
    return (h - m) / jnp.sqrt(v + 1e-5) * g + b


_SENT = 0x7FFFFFFF


def _knn_body(lohi_ref, prow_ref, posT_ref, brow_ref, bcol_ref, idx_ref, wbuf):
    t = pl.program_id(0)
    c0 = lohi_ref[t, 0]
    nc = lohi_ref[t, 1] - c0

    prow = prow_ref[...]
    sqr = jnp.sum(prow * prow, axis=1, keepdims=True)
    brow = brow_ref[...]
    rowid = t * TR + lax.broadcasted_iota(jnp.int32, (TR, 1), 0)

    def mat_body(l, _):
        c = c0 + l
        cs = pl.multiple_of(c * CW, CW)
        pcol = posT_ref[:, pl.ds(cs, CW)]
        dot = jnp.dot(prow, pcol, preferred_element_type=jnp.float32)
        sqc = jnp.sum(pcol * pcol, axis=0, keepdims=True)
        d2 = jnp.maximum(sqr + sqc - 2.0 * dot, 0.0)
        bcol = bcol_ref[:, pl.ds(cs, CW)]
        colid = c * CW + lax.broadcasted_iota(jnp.int32, (TR, CW), 1)
        valid = (brow == bcol) & (colid != rowid)
        lc = l * CW + lax.broadcasted_iota(jnp.int32, (TR, CW), 1)
        key = (lax.bitcast_convert_type(d2, jnp.int32) & jnp.int32(~0xFFF)) | lc
        ls = pl.multiple_of(l * CW, CW)
        wbuf[:, pl.ds(ls, CW)] = jnp.where(valid, key, _SENT)
        return 0

    lax.fori_loop(0, nc, mat_body, 0)

    base = c0 * CW
    last = jnp.full((TR, 1), -1, jnp.int32)
    for k in range(K):
        def scan_body(l, m, last=last):
            ls = pl.multiple_of(l * CW, CW)
            w = wbuf[:, pl.ds(ls, CW)]
            return jnp.minimum(m, jnp.min(
                jnp.where(w > last, w, _SENT), axis=1, keepdims=True))

        m = lax.fori_loop(0, nc, scan_body,
                          jnp.full((TR, 1), _SENT, jnp.int32))
        idx_ref[:, k:k + 1] = jnp.minimum((m & 0xFFF) + base, N - 1)
        last = m


def _knn_call(lohi, prow, posT, brow, bcol):
    return pl.pallas_call(
        _knn_body,
        grid=(NT_KNN,),
        in_specs=[
            pl.BlockSpec(memory_space=pltpu.SMEM),
            pl.BlockSpec((TR, 8), lambda t: (t, 0)),
            pl.BlockSpec((8, NP), lambda t: (0, 0)),
            pl.BlockSpec((TR, 1), lambda t: (t, 0)),
            pl.BlockSpec((1, NP), lambda t: (0, 0)),
        ],
        out_specs=pl.BlockSpec((TR, K), lambda t: (t, 0)),
        out_shape=jax.ShapeDtypeStruct((NP, K), jnp.int32),
        scratch_shapes=[pltpu.VMEM((TR, WCAP), jnp.int32)],
    )(lohi, prow, posT, brow, bcol)


def _enc_body(x_ref, w1_ref, b1_ref, w2_ref, b2_ref, g_ref, b_ref, o_ref):
    x = x_ref[...]
    h1 = jnp.maximum(
        jnp.dot(x, w1_ref[...], preferred_element_type=jnp.float32) + b1_ref[...],
        0.0)
    h = jnp.dot(h1, w2_ref[...], preferred_element_type=jnp.float32) + b2_ref[...]
    o_ref[...] = _ln(h, g_ref[...], b_ref[...])


def _enc_call(xin, w1, b1, w2, b2, g, b):
    return pl.pallas_call(
        _enc_body,
        grid=(NP // TRE,),
        in_specs=[
            pl.BlockSpec((TRE, DM), lambda t: (t, 0)),
            pl.BlockSpec((DM, 64), lambda t: (0, 0)),
            pl.BlockSpec((1, 64), lambda t: (0, 0)),
            pl.BlockSpec((64, DM), lambda t: (0, 0)),
            pl.BlockSpec((1, DM), lambda t: (0, 0)),
            pl.BlockSpec((1, DM), lambda t: (0, 0)),
            pl.BlockSpec((1, DM), lambda t: (0, 0)),
        ],
        out_specs=pl.BlockSpec((TRE, DM), lambda t: (t, 0)),
        out_shape=jax.ShapeDtypeStruct((NP, DM), jnp.float32),
    )(xin, w1, b1, w2, b2, g, b)


def _pre_body(h_ref, w_ref, adl_ref, S_ref, xw_ref, adst_ref):
    xw = jnp.dot(h_ref[...], w_ref[...], preferred_element_type=jnp.float32)
    xw_ref[...] = xw
    adst_ref[...] = jnp.dot(xw * adl_ref[...], S_ref[...],
                            preferred_element_type=jnp.float32)


def _pre_call(h, gat_w, att_dst_lane, S):
    return pl.pallas_call(
        _pre_body,
        grid=(NP // TRE,),
        in_specs=[
            pl.BlockSpec((TRE, DM), lambda t: (t, 0)),
            pl.BlockSpec((DM, DM), lambda t: (0, 0)),
            pl.BlockSpec((1, DM), lambda t: (0, 0)),
            pl.BlockSpec((DM, DM), lambda t: (0, 0)),
        ],
        out_specs=[
            pl.BlockSpec((TRE, DM), lambda t: (t, 0)),
            pl.BlockSpec((TRE, DM), lambda t: (t, 0)),
        ],
        out_shape=[
            jax.ShapeDtypeStruct((NP, DM), jnp.float32),
            jax.ShapeDtypeStruct((NP, DM), jnp.float32),
        ],
    )(h, gat_w, att_dst_lane, S)


NBUF = 4


def _sc_gather_body(table_hbm, idx_hbm, out_hbm, idx_v, rows_v,
                    g0, g1, g2, g3, ws):
    wid = lax.axis_index("s") * 2 + lax.axis_index("c")
    pltpu.sync_copy(idx_hbm.at[wid], idx_v)
    gsem = (g0, g1, g2, g3)

    def round_body(r, _):
        j0 = r * NBUF
        gathers = []
        for b in range(NBUF):
            gathers.append(pltpu.async_copy(
                table_hbm.at[idx_v.at[j0 + b]], rows_v.at[b], gsem[b]))
        writes = []
        for b in range(NBUF):
            gathers[b].wait()
            writes.append(pltpu.async_copy(
                rows_v.at[b],
                out_hbm.at[pl.ds(wid * EP + (j0 + b) * CH, CH)], ws))
        for w in writes:
            w.wait()
        return 0

    lax.fori_loop(0, NCH // NBUF, round_body, 0)


@functools.lru_cache(maxsize=1)
def _sc_gather_kernel():
    return pl.kernel(
        _sc_gather_body,
        mesh=plsc.VectorSubcoreMesh(core_axis_name="c", subcore_axis_name="s",
                                    num_cores=2),
        out_type=jax.ShapeDtypeStruct((BPAD, DM), jnp.float32),
        scratch_types=[
            pltpu.VMEM((NCH, CH), jnp.int32),
            pltpu.VMEM((NBUF, CH, DM), jnp.float32),
            pltpu.SemaphoreType.DMA,
            pltpu.SemaphoreType.DMA,
            pltpu.SemaphoreType.DMA,
            pltpu.SemaphoreType.DMA,
            pltpu.SemaphoreType.DMA,
        ],
    )


def _sc_gather(table, idx3):
    return _sc_gather_kernel()(table, idx3)


def _post_body(xwg_ref, h_ref, adst_ref, asl_ref, S_ref, gb_ref,
               l1g_ref, l1b_ref, w1_ref, b1_ref, w2_ref, b2_ref,
               l2g_ref, l2b_ref, o_ref):
    xwg = xwg_ref[...]
    asrc = jnp.dot(xwg * asl_ref[...], S_ref[...],
                   preferred_element_type=jnp.float32)
    e = asrc.reshape(TNP, K, DM) + adst_ref[...][:, None, :]
    e = jnp.where(e >= 0.0, e, 0.2 * e)
    m = jnp.max(e, axis=1, keepdims=True)
    ex = jnp.exp(e - m)
    s = jnp.sum(ex, axis=1, keepdims=True)
    alpha = ex / s
    agg = jnp.sum(alpha * xwg.reshape(TNP, K, DM), axis=1)
    u = _ln(h_ref[...] + agg + gb_ref[...], l1g_ref[...], l1b_ref[...])
    f1 = jnp.maximum(
        jnp.dot(u, w1_ref[...], preferred_element_type=jnp.float32) + b1_ref[...],
        0.0)
    f = jnp.dot(f1, w2_ref[...], preferred_element_type=jnp.float32) + b2_ref[...]
    o_ref[...] = _ln(u + f, l2g_ref[...], l2b_ref[...])


def _post_call(xwg, h, adst, att_src_lane, S, gat_b, l1g, l1b,
               ff_w1, ff_b1, ff_w2, ff_b2, l2g, l2b):
    c = lambda shape: pl.BlockSpec(shape, lambda t: (0, 0))
    return pl.pallas_call(
        _post_body,
        grid=(NP // TNP,),
        in_specs=[
            pl.BlockSpec((TNP * K, DM), lambda t: (t, 0)),
            pl.BlockSpec((TNP, DM), lambda t: (t, 0)),
            pl.BlockSpec((TNP, DM), lambda t: (t, 0)),
            c((1, DM)), c((DM, DM)), c((1, DM)),
            c((1, DM)), c((1, DM)),
            c((DM, 2 * DM)), c((1, 2 * DM)), c((2 * DM, DM)), c((1, DM)),
            c((1, DM)), c((1, DM)),
        ],
        out_specs=pl.BlockSpec((TNP, DM), lambda t: (t, 0)),
        out_shape=jax.ShapeDtypeStruct((NP, DM), jnp.float32),
    )(xwg, h, adst, att_src_lane, S, gat_b, l1g, l1b,
      ff_w1, ff_b1, ff_w2, ff_b2, l2g, l2b)


def _fin_body(h_ref, bcol_ref, g_ref, b_ref, cw_ref, cb_ref, o_ref,
              acc_s, cnt_s):
    t = pl.program_id(0)
    h = _ln(h_ref[...], g_ref[...], b_ref[...])
    giota = lax.broadcasted_iota(jnp.int32, (G, TRE), 0)
    M = (giota == bcol_ref[...]).astype(jnp.float32)

    @pl.when(t == 0)
    def _():
        acc_s[...] = jnp.zeros((G, DM), jnp.float32)
        cnt_s[...] = jnp.zeros((G, DM), jnp.float32)

    acc_s[...] += jnp.dot(M, h, preferred_element_type=jnp.float32)
    cnt_s[...] += jnp.sum(M, axis=1, keepdims=True) + jnp.zeros((G, DM), jnp.float32)

    @pl.when(t == NP // TRE - 1)
    def _():
        pooled = acc_s[...] / cnt_s[...]
        o_ref[...] = (jnp.dot(pooled, cw_ref[...],
                              preferred_element_type=jnp.float32) + cb_ref[...])


def _fin_call(h, bcol, ln_g, ln_b, cls_w, cls_b):
    return pl.pallas_call(
        _fin_body,
        grid=(NP // TRE,),
        in_specs=[
            pl.BlockSpec((TRE, DM), lambda t: (t, 0)),
            pl.BlockSpec((1, TRE), lambda t: (0, t)),
            pl.BlockSpec((1, DM), lambda t: (0, 0)),
            pl.BlockSpec((1, DM), lambda t: (0, 0)),
            pl.BlockSpec((DM, OUT), lambda t: (0, 0)),
            pl.BlockSpec((1, OUT), lambda t: (0, 0)),
        ],
        out_specs=pl.BlockSpec((G, OUT), lambda t: (0, 0)),
        out_shape=jax.ShapeDtypeStruct((G, OUT), jnp.float32),
        scratch_shapes=[
            pltpu.VMEM((G, DM), jnp.float32),
            pltpu.VMEM((G, DM), jnp.float32),
        ],
    )(h, bcol, ln_g, ln_b, cls_w, cls_b)


_S_HEAD = jnp.asarray(np.kron(np.eye(HEADS), np.ones((FEAT, FEAT))), jnp.float32)


def kernel(x, pos, batch, params):
    p = params
    batch_i = batch.astype(jnp.int32)

    xin = jnp.concatenate([x, pos], axis=1)
    xin_p = jnp.pad(xin, ((0, NP - N), (0, 0)))
    pos8 = jnp.pad(pos, ((0, NP - N), (0, 5)))
    posT = pos8.T
    brow = jnp.pad(batch_i, (0, NP - N), constant_values=-1).reshape(NP, 1)
    bcol = jnp.pad(batch_i, (0, NP - N), constant_values=-2).reshape(1, NP)

    se = jnp.searchsorted(batch_i, jnp.arange(G + 1, dtype=jnp.int32))
    starts, ends = se[:G], se[1:]
    tfirst = jnp.minimum(jnp.arange(NT_KNN) * TR, N - 1)
    tlast = jnp.minimum(jnp.arange(1, NT_KNN + 1) * TR - 1, N - 1)
    lo = starts[batch_i[tfirst]]
    hi = ends[batch_i[tlast]]
    lo_c = lo // CW
    hi_c = jnp.minimum((hi + CW - 1) // CW, lo_c + MAXC)
    lohi = jnp.stack([lo_c, hi_c], axis=1).astype(jnp.int32)

    idx_full = _knn_call(lohi, pos8, posT, brow, bcol)
    src = idx_full[:N].reshape(-1)
    src3 = jnp.pad(src, (0, BPAD - N * K)).reshape(NW, NCH, CH)
    src3 = src3 + (jnp.arange(NW, dtype=jnp.int32) % 2 * NP).reshape(NW, 1, 1)

    h = _enc_call(xin_p, p["enc_w1"], p["enc_b1"].reshape(1, 64),
                  p["enc_w2"], p["enc_b2"].reshape(1, DM),
                  p["enc_ln_g"].reshape(1, DM), p["enc_ln_b"].reshape(1, DM))

    for lp in p["layers"]:
        xw, adst = _pre_call(h, lp["gat_w"],
                             lp["att_dst"].reshape(1, DM), _S_HEAD)
        xwg = _sc_gather(jnp.concatenate([xw, xw], axis=0), src3)
        h = _post_call(xwg, h, adst, lp["att_src"].reshape(1, DM), _S_HEAD,
                       lp["gat_b"].reshape(1, DM),
                       lp["ln1_g"].reshape(1, DM), lp["ln1_b"].reshape(1, DM),
                       lp["ff_w1"], lp["ff_b1"].reshape(1, 2 * DM),
                       lp["ff_w2"], lp["ff_b2"].reshape(1, DM),
                       lp["ln2_g"].reshape(1, DM), lp["ln2_b"].reshape(1, DM))

    return _fin_call(h, bcol, p["ln_g"].reshape(1, DM), p["ln_b"].reshape(1, DM),
                     p["cls_w"], p["cls_b"].reshape(1, OUT))


# speedup vs baseline: 1.0379x; 1.0379x over previous
import functools

import numpy as np
import jax
import jax.numpy as jnp
from jax import lax
from jax.experimental import pallas as pl
from jax.experimental.pallas import tpu as pltpu
from jax.experimental.pallas import tpu_sc as plsc

N = 10000
K = 32
G = 16
DM = 128
HEADS = 8
FEAT = 16
OUT = 10

NP = 10240
TR = 256
CW = 512
WCAP = 4096
NT_KNN = NP // TR
MAXC = WCAP // CW

TRE = 512
TNP = 128
NW = 32
CH = 128
BPAD = NP * K
EP = BPAD // NW
NCH = EP // CH

_INF = float("inf")
_BIGI = 2**30


def _ln(h, g, b):
    m = jnp.mean(h, axis=-1, keepdims=True)
    v = jnp.mean((h - m) * (h - m), axis=-1, keepdims=True)
    return (h - m) / jnp.sqrt(v + 1e-5) * g + b


_SENT = 0x7FFFFFFF


def _knn_body(lohi_ref, prow_ref, posT_ref, brow_ref, bcol_ref, idx_ref, wbuf):
    t = pl.program_id(0)
    c0 = lohi_ref[t, 0]
    nc = lohi_ref[t, 1] - c0

    prow = prow_ref[...]
    sqr = jnp.sum(prow * prow, axis=1, keepdims=True)
    brow = brow_ref[...]
    rowid = t * TR + lax.broadcasted_iota(jnp.int32, (TR, 1), 0)

    def mat_body(l, _):
        c = c0 + l
        cs = pl.multiple_of(c * CW, CW)
        pcol = posT_ref[:, pl.ds(cs, CW)]
        dot = jnp.dot(prow, pcol, preferred_element_type=jnp.float32)
        sqc = jnp.sum(pcol * pcol, axis=0, keepdims=True)
        d2 = jnp.maximum(sqr + sqc - 2.0 * dot, 0.0)
        bcol = bcol_ref[:, pl.ds(cs, CW)]
        colid = c * CW + lax.broadcasted_iota(jnp.int32, (TR, CW), 1)
        valid = (brow == bcol) & (colid != rowid)
        lc = l * CW + lax.broadcasted_iota(jnp.int32, (TR, CW), 1)
        key = (lax.bitcast_convert_type(d2, jnp.int32) & jnp.int32(~0xFFF)) | lc
        ls = pl.multiple_of(l * CW, CW)
        wbuf[:, pl.ds(ls, CW)] = jnp.where(valid, key, _SENT)
        return 0

    lax.fori_loop(0, nc, mat_body, 0)

    base = c0 * CW
    last = jnp.full((TR, 1), -1, jnp.int32)
    for k in range(K):
        def scan_body(l, m, last=last):
            ls = pl.multiple_of(l * CW, CW)
            w = wbuf[:, pl.ds(ls, CW)]
            return jnp.minimum(m, jnp.min(
                jnp.where(w > last, w, _SENT), axis=1, keepdims=True))

        m = lax.fori_loop(0, nc, scan_body,
                          jnp.full((TR, 1), _SENT, jnp.int32))
        idx_ref[:, k:k + 1] = jnp.minimum((m & 0xFFF) + base, N - 1)
        last = m


def _knn_call(lohi, prow, posT, brow, bcol):
    return pl.pallas_call(
        _knn_body,
        grid=(NT_KNN,),
        in_specs=[
            pl.BlockSpec(memory_space=pltpu.SMEM),
            pl.BlockSpec((TR, 8), lambda t: (t, 0)),
            pl.BlockSpec((8, NP), lambda t: (0, 0)),
            pl.BlockSpec((TR, 1), lambda t: (t, 0)),
            pl.BlockSpec((1, NP), lambda t: (0, 0)),
        ],
        out_specs=pl.BlockSpec((TR, K), lambda t: (t, 0)),
        out_shape=jax.ShapeDtypeStruct((NP, K), jnp.int32),
        scratch_shapes=[pltpu.VMEM((TR, WCAP), jnp.int32)],
    )(lohi, prow, posT, brow, bcol)


def _enc_body(x_ref, w1_ref, b1_ref, w2_ref, b2_ref, g_ref, b_ref, o_ref):
    x = x_ref[...]
    h1 = jnp.maximum(
        jnp.dot(x, w1_ref[...], preferred_element_type=jnp.float32) + b1_ref[...],
        0.0)
    h = jnp.dot(h1, w2_ref[...], preferred_element_type=jnp.float32) + b2_ref[...]
    o_ref[...] = _ln(h, g_ref[...], b_ref[...])


def _enc_call(xin, w1, b1, w2, b2, g, b):
    return pl.pallas_call(
        _enc_body,
        grid=(NP // TRE,),
        in_specs=[
            pl.BlockSpec((TRE, DM), lambda t: (t, 0)),
            pl.BlockSpec((DM, 64), lambda t: (0, 0)),
            pl.BlockSpec((1, 64), lambda t: (0, 0)),
            pl.BlockSpec((64, DM), lambda t: (0, 0)),
            pl.BlockSpec((1, DM), lambda t: (0, 0)),
            pl.BlockSpec((1, DM), lambda t: (0, 0)),
            pl.BlockSpec((1, DM), lambda t: (0, 0)),
        ],
        out_specs=pl.BlockSpec((TRE, DM), lambda t: (t, 0)),
        out_shape=jax.ShapeDtypeStruct((NP, DM), jnp.float32),
    )(xin, w1, b1, w2, b2, g, b)


def _pre_body(h_ref, w_ref, adl_ref, S_ref, xw_ref, adst_ref):
    xw = jnp.dot(h_ref[...], w_ref[...], preferred_element_type=jnp.float32)
    xw_ref[...] = xw
    adst_ref[...] = jnp.dot(xw * adl_ref[...], S_ref[...],
                            preferred_element_type=jnp.float32)


def _pre_call(h, gat_w, att_dst_lane, S):
    return pl.pallas_call(
        _pre_body,
        grid=(NP // TRE,),
        in_specs=[
            pl.BlockSpec((TRE, DM), lambda t: (t, 0)),
            pl.BlockSpec((DM, DM), lambda t: (0, 0)),
            pl.BlockSpec((1, DM), lambda t: (0, 0)),
            pl.BlockSpec((DM, DM), lambda t: (0, 0)),
        ],
        out_specs=[
            pl.BlockSpec((TRE, DM), lambda t: (t, 0)),
            pl.BlockSpec((TRE, DM), lambda t: (t, 0)),
        ],
        out_shape=[
            jax.ShapeDtypeStruct((NP, DM), jnp.float32),
            jax.ShapeDtypeStruct((NP, DM), jnp.float32),
        ],
    )(h, gat_w, att_dst_lane, S)


NBUF = 4
NCHT = BPAD // CH
NCH0 = 120
NCH1 = (NCHT - 16 * NCH0) // 16
CORE0_TOT = 16 * NCH0


def _sc_gather_body(table_hbm, idx_hbm, out_hbm, idx_v, rows_v,
                    g0, g1, g2, g3, ws):
    s = lax.axis_index("s")
    c = lax.axis_index("c")
    start = jnp.where(c == 0, s * NCH0, CORE0_TOT + s * NCH1)
    cnt = jnp.where(c == 0, NCH0, NCH1)
    gsem = (g0, g1, g2, g3)

    @pl.when(c == 0)
    def _():
        pltpu.sync_copy(idx_hbm.at[pl.ds(s * NCH0, NCH0)], idx_v)

    @pl.when(c == 1)
    def _():
        pltpu.sync_copy(idx_hbm.at[pl.ds(CORE0_TOT + s * NCH1, NCH1)],
                        idx_v.at[pl.ds(0, NCH1)])

    def round_body(r, _):
        j0 = r * NBUF
        gathers = []
        for b in range(NBUF):
            gathers.append(pltpu.async_copy(
                table_hbm.at[idx_v.at[j0 + b]], rows_v.at[b], gsem[b]))
        writes = []
        for b in range(NBUF):
            gathers[b].wait()
            writes.append(pltpu.async_copy(
                rows_v.at[b],
                out_hbm.at[pl.ds((start + j0 + b) * CH, CH)], ws))
        for w in writes:
            w.wait()
        return 0

    lax.fori_loop(0, cnt // NBUF, round_body, 0)


@functools.lru_cache(maxsize=1)
def _sc_gather_kernel():
    return pl.kernel(
        _sc_gather_body,
        mesh=plsc.VectorSubcoreMesh(core_axis_name="c", subcore_axis_name="s",
                                    num_cores=2),
        out_type=jax.ShapeDtypeStruct((BPAD, DM), jnp.float32),
        scratch_types=[
            pltpu.VMEM((NCH0, CH), jnp.int32),
            pltpu.VMEM((NBUF, CH, DM), jnp.float32),
            pltpu.SemaphoreType.DMA,
            pltpu.SemaphoreType.DMA,
            pltpu.SemaphoreType.DMA,
            pltpu.SemaphoreType.DMA,
            pltpu.SemaphoreType.DMA,
        ],
    )


def _sc_gather(table, idx3):
    return _sc_gather_kernel()(table, idx3)


def _post_body(xwg_ref, h_ref, adst_ref, asl_ref, S_ref, gb_ref,
               l1g_ref, l1b_ref, w1_ref, b1_ref, w2_ref, b2_ref,
               l2g_ref, l2b_ref, o_ref):
    xwg = xwg_ref[...]
    asrc = jnp.dot(xwg * asl_ref[...], S_ref[...],
                   preferred_element_type=jnp.float32)
    e = asrc.reshape(TNP, K, DM) + adst_ref[...][:, None, :]
    e = jnp.where(e >= 0.0, e, 0.2 * e)
    m = jnp.max(e, axis=1, keepdims=True)
    ex = jnp.exp(e - m)
    s = jnp.sum(ex, axis=1, keepdims=True)
    alpha = ex / s
    agg = jnp.sum(alpha * xwg.reshape(TNP, K, DM), axis=1)
    u = _ln(h_ref[...] + agg + gb_ref[...], l1g_ref[...], l1b_ref[...])
    f1 = jnp.maximum(
        jnp.dot(u, w1_ref[...], preferred_element_type=jnp.float32) + b1_ref[...],
        0.0)
    f = jnp.dot(f1, w2_ref[...], preferred_element_type=jnp.float32) + b2_ref[...]
    o_ref[...] = _ln(u + f, l2g_ref[...], l2b_ref[...])


def _post_call(xwg, h, adst, att_src_lane, S, gat_b, l1g, l1b,
               ff_w1, ff_b1, ff_w2, ff_b2, l2g, l2b):
    c = lambda shape: pl.BlockSpec(shape, lambda t: (0, 0))
    return pl.pallas_call(
        _post_body,
        grid=(NP // TNP,),
        in_specs=[
            pl.BlockSpec((TNP * K, DM), lambda t: (t, 0)),
            pl.BlockSpec((TNP, DM), lambda t: (t, 0)),
            pl.BlockSpec((TNP, DM), lambda t: (t, 0)),
            c((1, DM)), c((DM, DM)), c((1, DM)),
            c((1, DM)), c((1, DM)),
            c((DM, 2 * DM)), c((1, 2 * DM)), c((2 * DM, DM)), c((1, DM)),
            c((1, DM)), c((1, DM)),
        ],
        out_specs=pl.BlockSpec((TNP, DM), lambda t: (t, 0)),
        out_shape=jax.ShapeDtypeStruct((NP, DM), jnp.float32),
    )(xwg, h, adst, att_src_lane, S, gat_b, l1g, l1b,
      ff_w1, ff_b1, ff_w2, ff_b2, l2g, l2b)


def _fin_body(h_ref, bcol_ref, g_ref, b_ref, cw_ref, cb_ref, o_ref,
              acc_s, cnt_s):
    t = pl.program_id(0)
    h = _ln(h_ref[...], g_ref[...], b_ref[...])
    giota = lax.broadcasted_iota(jnp.int32, (G, TRE), 0)
    M = (giota == bcol_ref[...]).astype(jnp.float32)

    @pl.when(t == 0)
    def _():
        acc_s[...] = jnp.zeros((G, DM), jnp.float32)
        cnt_s[...] = jnp.zeros((G, DM), jnp.float32)

    acc_s[...] += jnp.dot(M, h, preferred_element_type=jnp.float32)
    cnt_s[...] += jnp.sum(M, axis=1, keepdims=True) + jnp.zeros((G, DM), jnp.float32)

    @pl.when(t == NP // TRE - 1)
    def _():
        pooled = acc_s[...] / cnt_s[...]
        o_ref[...] = (jnp.dot(pooled, cw_ref[...],
                              preferred_element_type=jnp.float32) + cb_ref[...])


def _fin_call(h, bcol, ln_g, ln_b, cls_w, cls_b):
    return pl.pallas_call(
        _fin_body,
        grid=(NP // TRE,),
        in_specs=[
            pl.BlockSpec((TRE, DM), lambda t: (t, 0)),
            pl.BlockSpec((1, TRE), lambda t: (0, t)),
            pl.BlockSpec((1, DM), lambda t: (0, 0)),
            pl.BlockSpec((1, DM), lambda t: (0, 0)),
            pl.BlockSpec((DM, OUT), lambda t: (0, 0)),
            pl.BlockSpec((1, OUT), lambda t: (0, 0)),
        ],
        out_specs=pl.BlockSpec((G, OUT), lambda t: (0, 0)),
        out_shape=jax.ShapeDtypeStruct((G, OUT), jnp.float32),
        scratch_shapes=[
            pltpu.VMEM((G, DM), jnp.float32),
            pltpu.VMEM((G, DM), jnp.float32),
        ],
    )(h, bcol, ln_g, ln_b, cls_w, cls_b)


_S_HEAD = jnp.asarray(np.kron(np.eye(HEADS), np.ones((FEAT, FEAT))), jnp.float32)


def kernel(x, pos, batch, params):
    p = params
    batch_i = batch.astype(jnp.int32)

    xin = jnp.concatenate([x, pos], axis=1)
    xin_p = jnp.pad(xin, ((0, NP - N), (0, 0)))
    pos8 = jnp.pad(pos, ((0, NP - N), (0, 5)))
    posT = pos8.T
    brow = jnp.pad(batch_i, (0, NP - N), constant_values=-1).reshape(NP, 1)
    bcol = jnp.pad(batch_i, (0, NP - N), constant_values=-2).reshape(1, NP)

    se = jnp.searchsorted(batch_i, jnp.arange(G + 1, dtype=jnp.int32))
    starts, ends = se[:G], se[1:]
    tfirst = jnp.minimum(jnp.arange(NT_KNN) * TR, N - 1)
    tlast = jnp.minimum(jnp.arange(1, NT_KNN + 1) * TR - 1, N - 1)
    lo = starts[batch_i[tfirst]]
    hi = ends[batch_i[tlast]]
    lo_c = lo // CW
    hi_c = jnp.minimum((hi + CW - 1) // CW, lo_c + MAXC)
    lohi = jnp.stack([lo_c, hi_c], axis=1).astype(jnp.int32)

    idx_full = _knn_call(lohi, pos8, posT, brow, bcol)
    src = idx_full[:N].reshape(-1)
    src3 = jnp.pad(src, (0, BPAD - N * K)).reshape(NCHT, CH)

    h = _enc_call(xin_p, p["enc_w1"], p["enc_b1"].reshape(1, 64),
                  p["enc_w2"], p["enc_b2"].reshape(1, DM),
                  p["enc_ln_g"].reshape(1, DM), p["enc_ln_b"].reshape(1, DM))

    for lp in p["layers"]:
        xw, adst = _pre_call(h, lp["gat_w"],
                             lp["att_dst"].reshape(1, DM), _S_HEAD)
        xwg = _sc_gather(xw, src3)
        h = _post_call(xwg, h, adst, lp["att_src"].reshape(1, DM), _S_HEAD,
                       lp["gat_b"].reshape(1, DM),
                       lp["ln1_g"].reshape(1, DM), lp["ln1_b"].reshape(1, DM),
                       lp["ff_w1"], lp["ff_b1"].reshape(1, 2 * DM),
                       lp["ff_w2"], lp["ff_b2"].reshape(1, DM),
                       lp["ln2_g"].reshape(1, DM), lp["ln2_b"].reshape(1, DM))

    return _fin_call(h, bcol, p["ln_g"].reshape(1, DM), p["ln_b"].reshape(1, DM),
                     p["cls_w"], p["cls_b"].reshape(1, OUT))
